# baseline (speedup 1.0000x reference)
import functools

import jax
import jax.numpy as jnp
from jax import lax
from jax.experimental import pallas as pl
from jax.experimental.pallas import tpu as pltpu
from jax.experimental.pallas import tpu_sc as plsc


def _make_sc_gather(n_rows, dim, n_workers, chunk):
    mesh = plsc.VectorSubcoreMesh(core_axis_name="c", subcore_axis_name="s")
    per_w = n_rows // n_workers
    n_chunks = per_w // chunk

    @functools.partial(
        pl.kernel,
        out_type=jax.ShapeDtypeStruct((n_rows, dim), jnp.float32),
        mesh=mesh,
        scratch_types=[
            pltpu.VMEM((per_w,), jnp.int32),
            pltpu.VMEM((chunk, dim), jnp.float32),
            pltpu.VMEM((chunk, dim), jnp.float32),
            pltpu.SemaphoreType.DMA,
            pltpu.SemaphoreType.DMA,
        ],
    )
    def gather(x_hbm, ids_hbm, xs_hbm, idx_v, rows0, rows1, sem0, sem1):
        nc = lax.axis_size("c")
        wid = lax.axis_index("s") * nc + lax.axis_index("c")
        base_w = wid * per_w
        pltpu.sync_copy(ids_hbm.at[pl.ds(base_w, per_w)], idx_v)
        bufs = (rows0, rows1)
        sems = (sem0, sem1)
        copies = [None] * n_chunks
        for ci in range(n_chunks):
            copies[ci] = pltpu.async_copy(
                x_hbm.at[idx_v.at[pl.ds(ci * chunk, chunk)]],
                bufs[ci % 2], sems[ci % 2])
            if ci > 0:
                copies[ci - 1].wait()
                pltpu.sync_copy(bufs[(ci - 1) % 2],
                                xs_hbm.at[pl.ds(base_w + (ci - 1) * chunk,
                                                chunk)])
        copies[n_chunks - 1].wait()
        pltpu.sync_copy(bufs[(n_chunks - 1) % 2],
                        xs_hbm.at[pl.ds(base_w + (n_chunks - 1) * chunk,
                                        chunk)])

    return gather


def _make_sc_combine(n_tok, dim, topk, n_workers, chunk):
    mesh = plsc.VectorSubcoreMesh(core_axis_name="c", subcore_axis_name="s")
    per_w = n_tok // n_workers
    n_chunks = per_w // chunk
    npair = chunk * topk

    @functools.partial(
        pl.kernel,
        out_type=jax.ShapeDtypeStruct((n_tok, dim), jnp.float32),
        mesh=mesh,
        scratch_types=[
            pltpu.VMEM((per_w * topk,), jnp.int32),
            pltpu.VMEM((npair, dim), jnp.float32),
            pltpu.VMEM((npair, dim), jnp.float32),
            pltpu.VMEM((chunk, dim), jnp.float32),
            pltpu.SemaphoreType.DMA,
            pltpu.SemaphoreType.DMA,
        ],
    )
    def combine(ys_hbm, pos_hbm, out_hbm, idx_v, rows0, rows1, acc_v,
                sem0, sem1):
        nc = lax.axis_size("c")
        wid = lax.axis_index("s") * nc + lax.axis_index("c")
        base_t = wid * per_w
        pltpu.sync_copy(pos_hbm.at[pl.ds(base_t * topk, per_w * topk)], idx_v)
        bufs = (rows0, rows1)
        sems = (sem0, sem1)
        copies = [None] * n_chunks

        def compute(rows_v, ci):
            def body(j, _):
                off = j * 16
                for t in range(chunk):
                    v = rows_v[t * topk, pl.ds(off, 16)]
                    for k in range(1, topk):
                        v = v + rows_v[t * topk + k, pl.ds(off, 16)]
                    acc_v[t, pl.ds(off, 16)] = v
                return 0

            lax.fori_loop(0, dim // 16, body, 0)
            pltpu.sync_copy(acc_v,
                            out_hbm.at[pl.ds(base_t + ci * chunk, chunk)])

        for ci in range(n_chunks):
            copies[ci] = pltpu.async_copy(
                ys_hbm.at[idx_v.at[pl.ds(ci * npair, npair)]],
                bufs[ci % 2], sems[ci % 2])
            if ci > 0:
                copies[ci - 1].wait()
                compute(bufs[(ci - 1) % 2], ci - 1)
        copies[n_chunks - 1].wait()
        compute(bufs[(n_chunks - 1) % 2], n_chunks - 1)

    return combine


def _grouped_ffn_body(bids_ref, eids_ref, valids_ref, offs_ref,
                      xs_ref, g_ref, u_ref, d_ref, ws_ref, out_ref,
                      *, blk, inter):
    g = pl.program_id(0)
    b = bids_ref[g]
    e = eids_ref[g]
    valid = valids_ref[g]
    row0 = b * blk
    lo = jnp.clip(offs_ref[e] - row0, 0, blk)
    hi = jnp.clip(offs_ref[e + 1] - row0, 0, blk)
    hi = jnp.where(valid > 0, hi, lo)

    xb = xs_ref[...]
    gg = jax.lax.dot_general(xb, g_ref[...], (((1,), (1,)), ((), ())),
                             preferred_element_type=jnp.float32)
    uu = jax.lax.dot_general(xb, u_ref[...], (((1,), (1,)), ((), ())),
                             preferred_element_type=jnp.float32)
    h = gg * jax.nn.sigmoid(gg) * uu

    rows = jax.lax.broadcasted_iota(jnp.int32, (blk, 1), 0)
    inrange = (rows >= lo) & (rows < hi)
    ws = ws_ref[...]
    h = h * jnp.where(inrange, ws, 0.0)

    y = jax.lax.dot_general(h, d_ref[...],
                            (((1,), (1,)), ((), ())),
                            preferred_element_type=jnp.float32)

    prev = bids_ref[jnp.maximum(g - 1, 0)]
    is_first = (g == 0) | (prev != b)

    @pl.when(is_first)
    def _init():
        out_ref[...] = y

    @pl.when(jnp.logical_not(is_first))
    def _acc():
        out_ref[...] += y


def kernel(x, token_to_expert_indices, weights, gate_proj_weight,
           up_proj_weight, down_proj_weight):
    T, DIM = x.shape
    E, INTER, _ = gate_proj_weight.shape
    TOPK = token_to_expert_indices.shape[1]
    TK = T * TOPK
    BLK = 256
    NB = TK // BLK
    G = NB + E - 1

    e_flat = token_to_expert_indices.reshape(-1).astype(jnp.int32)
    w_flat = weights.reshape(-1)
    iota = jnp.arange(TK, dtype=jnp.int32)
    perm = jnp.argsort(e_flat, stable=True).astype(jnp.int32)
    e_sorted = e_flat[perm]
    pos = jnp.zeros((TK,), jnp.int32).at[perm].set(iota)
    token_ids = perm // TOPK
    w_sorted = w_flat[perm]
    offsets = jnp.searchsorted(
        e_sorted, jnp.arange(E + 1, dtype=jnp.int32)).astype(jnp.int32)
    counts = offsets[1:] - offsets[:E]

    b0 = offsets[:E] // BLK
    b1 = jnp.maximum(offsets[1:] - 1, 0) // BLK
    nb_e = jnp.where(counts > 0, b1 - b0 + 1, 0)
    cum = jnp.cumsum(nb_e)
    gs = jnp.arange(G, dtype=jnp.int32)
    eids = jnp.searchsorted(cum, gs, side="right").astype(jnp.int32)
    valids = (gs < cum[E - 1]).astype(jnp.int32)
    eids = jnp.clip(eids, 0, E - 1)
    start = jnp.concatenate([jnp.zeros((1,), jnp.int32),
                             cum.astype(jnp.int32)])[eids]
    bids = b0[eids] + (gs - start)
    bids = jnp.where(valids > 0, bids, NB - 1).astype(jnp.int32)

    gather = _make_sc_gather(TK, DIM, 32, 16)
    xs = gather(x, token_ids)

    w2 = w_sorted.reshape(TK, 1)

    grid_spec = pltpu.PrefetchScalarGridSpec(
        num_scalar_prefetch=4,
        grid=(G,),
        in_specs=[
            pl.BlockSpec((BLK, DIM), lambda g, bids, eids, valids, offs: (bids[g], 0)),
            pl.BlockSpec((None, INTER, DIM), lambda g, bids, eids, valids, offs: (eids[g], 0, 0)),
            pl.BlockSpec((None, INTER, DIM), lambda g, bids, eids, valids, offs: (eids[g], 0, 0)),
            pl.BlockSpec((None, DIM, INTER), lambda g, bids, eids, valids, offs: (eids[g], 0, 0)),
            pl.BlockSpec((BLK, 1), lambda g, bids, eids, valids, offs: (bids[g], 0)),
        ],
        out_specs=pl.BlockSpec((BLK, DIM), lambda g, bids, eids, valids, offs: (bids[g], 0)),
    )
    ys = pl.pallas_call(
        functools.partial(_grouped_ffn_body, blk=BLK, inter=INTER),
        grid_spec=grid_spec,
        out_shape=jax.ShapeDtypeStruct((TK, DIM), jnp.float32),
        compiler_params=pltpu.CompilerParams(
            dimension_semantics=("arbitrary",),
        ),
    )(bids, eids, valids, offsets, xs, gate_proj_weight, up_proj_weight,
      down_proj_weight, w2)

    combine = _make_sc_combine(T, DIM, TOPK, 32, 8)
    out = combine(ys, pos)
    return out


# speedup vs baseline: 1.0047x; 1.0047x over previous
import functools

import jax
import jax.numpy as jnp
from jax import lax
from jax.experimental import pallas as pl
from jax.experimental.pallas import tpu as pltpu
from jax.experimental.pallas import tpu_sc as plsc


def _make_sc_gather(n_rows, dim, n_workers, chunk):
    mesh = plsc.VectorSubcoreMesh(core_axis_name="c", subcore_axis_name="s")
    per_w = n_rows // n_workers
    n_chunks = per_w // chunk

    @functools.partial(
        pl.kernel,
        out_type=jax.ShapeDtypeStruct((n_rows, dim), jnp.float32),
        mesh=mesh,
        scratch_types=[
            pltpu.VMEM((per_w,), jnp.int32),
            pltpu.VMEM((chunk, dim), jnp.float32),
            pltpu.VMEM((chunk, dim), jnp.float32),
            pltpu.SemaphoreType.DMA,
            pltpu.SemaphoreType.DMA,
            pltpu.SemaphoreType.DMA,
            pltpu.SemaphoreType.DMA,
        ],
    )
    def gather(x_hbm, ids_hbm, xs_hbm, idx_v, rows0, rows1,
               gsem0, gsem1, ssem0, ssem1):
        nc = lax.axis_size("c")
        wid = lax.axis_index("s") * nc + lax.axis_index("c")
        base_w = wid * per_w
        pltpu.sync_copy(ids_hbm.at[pl.ds(base_w, per_w)], idx_v)
        bufs = (rows0, rows1)
        gsems = (gsem0, gsem1)
        ssems = (ssem0, ssem1)
        gcp = [None] * n_chunks
        scp = [None] * n_chunks
        for ci in range(n_chunks):
            if ci >= 2:
                scp[ci - 2].wait()
            gcp[ci] = pltpu.async_copy(
                x_hbm.at[idx_v.at[pl.ds(ci * chunk, chunk)]],
                bufs[ci % 2], gsems[ci % 2])
            if ci > 0:
                gcp[ci - 1].wait()
                scp[ci - 1] = pltpu.async_copy(
                    bufs[(ci - 1) % 2],
                    xs_hbm.at[pl.ds(base_w + (ci - 1) * chunk, chunk)],
                    ssems[(ci - 1) % 2])
        last = n_chunks - 1
        gcp[last].wait()
        scp[last] = pltpu.async_copy(
            bufs[last % 2],
            xs_hbm.at[pl.ds(base_w + last * chunk, chunk)],
            ssems[last % 2])
        scp[last - 1].wait()
        scp[last].wait()

    return gather


def _make_sc_combine(n_tok, dim, topk, n_workers, chunk):
    mesh = plsc.VectorSubcoreMesh(core_axis_name="c", subcore_axis_name="s")
    per_w = n_tok // n_workers
    n_chunks = per_w // chunk
    npair = chunk * topk

    @functools.partial(
        pl.kernel,
        out_type=jax.ShapeDtypeStruct((n_tok, dim), jnp.float32),
        mesh=mesh,
        scratch_types=[
            pltpu.VMEM((per_w * topk,), jnp.int32),
            pltpu.VMEM((npair, dim), jnp.float32),
            pltpu.VMEM((npair, dim), jnp.float32),
            pltpu.VMEM((chunk, dim), jnp.float32),
            pltpu.VMEM((chunk, dim), jnp.float32),
            pltpu.SemaphoreType.DMA,
            pltpu.SemaphoreType.DMA,
            pltpu.SemaphoreType.DMA,
            pltpu.SemaphoreType.DMA,
        ],
    )
    def combine(ys_hbm, pos_hbm, out_hbm, idx_v, rows0, rows1, acc0, acc1,
                gsem0, gsem1, ssem0, ssem1):
        nc = lax.axis_size("c")
        wid = lax.axis_index("s") * nc + lax.axis_index("c")
        base_t = wid * per_w
        pltpu.sync_copy(pos_hbm.at[pl.ds(base_t * topk, per_w * topk)], idx_v)
        bufs = (rows0, rows1)
        accs = (acc0, acc1)
        gsems = (gsem0, gsem1)
        ssems = (ssem0, ssem1)
        gcp = [None] * n_chunks
        scp = [None] * n_chunks

        def compute(rows_v, acc_v, ci):
            def body(j, _):
                off = j * 16
                for t in range(chunk):
                    v = rows_v[t * topk, pl.ds(off, 16)]
                    for k in range(1, topk):
                        v = v + rows_v[t * topk + k, pl.ds(off, 16)]
                    acc_v[t, pl.ds(off, 16)] = v
                return 0

            lax.fori_loop(0, dim // 16, body, 0)
            return pltpu.async_copy(
                acc_v, out_hbm.at[pl.ds(base_t + ci * chunk, chunk)],
                ssems[ci % 2])

        for ci in range(n_chunks):
            gcp[ci] = pltpu.async_copy(
                ys_hbm.at[idx_v.at[pl.ds(ci * npair, npair)]],
                bufs[ci % 2], gsems[ci % 2])
            if ci > 0:
                gcp[ci - 1].wait()
                if ci >= 3:
                    scp[ci - 3].wait()
                scp[ci - 1] = compute(bufs[(ci - 1) % 2],
                                      accs[(ci - 1) % 2], ci - 1)
        last = n_chunks - 1
        gcp[last].wait()
        if n_chunks >= 3:
            scp[last - 2].wait()
        scp[last] = compute(bufs[last % 2], accs[last % 2], last)
        if n_chunks >= 2:
            scp[last - 1].wait()
        scp[last].wait()

    return combine


def _grouped_ffn_body(bids_ref, eids_ref, valids_ref, offs_ref,
                      xs_ref, g_ref, u_ref, d_ref, ws_ref, out_ref,
                      *, blk, inter):
    g = pl.program_id(0)
    b = bids_ref[g]
    e = eids_ref[g]
    valid = valids_ref[g]
    row0 = b * blk
    lo = jnp.clip(offs_ref[e] - row0, 0, blk)
    hi = jnp.clip(offs_ref[e + 1] - row0, 0, blk)
    hi = jnp.where(valid > 0, hi, lo)

    xb = xs_ref[...]
    gg = jax.lax.dot_general(xb, g_ref[...], (((1,), (1,)), ((), ())),
                             preferred_element_type=jnp.float32)
    uu = jax.lax.dot_general(xb, u_ref[...], (((1,), (1,)), ((), ())),
                             preferred_element_type=jnp.float32)
    h = gg * jax.nn.sigmoid(gg) * uu

    rows = jax.lax.broadcasted_iota(jnp.int32, (blk, 1), 0)
    inrange = (rows >= lo) & (rows < hi)
    ws = ws_ref[...]
    h = h * jnp.where(inrange, ws, 0.0)

    y = jax.lax.dot_general(h, d_ref[...],
                            (((1,), (1,)), ((), ())),
                            preferred_element_type=jnp.float32)

    prev = bids_ref[jnp.maximum(g - 1, 0)]
    is_first = (g == 0) | (prev != b)

    @pl.when(is_first)
    def _init():
        out_ref[...] = y

    @pl.when(jnp.logical_not(is_first))
    def _acc():
        out_ref[...] += y


def kernel(x, token_to_expert_indices, weights, gate_proj_weight,
           up_proj_weight, down_proj_weight):
    T, DIM = x.shape
    E, INTER, _ = gate_proj_weight.shape
    TOPK = token_to_expert_indices.shape[1]
    TK = T * TOPK
    BLK = 256
    NB = TK // BLK
    G = NB + E - 1

    e_flat = token_to_expert_indices.reshape(-1).astype(jnp.int32)
    w_flat = weights.reshape(-1)
    iota = jnp.arange(TK, dtype=jnp.int32)
    perm = jnp.argsort(e_flat, stable=True).astype(jnp.int32)
    e_sorted = e_flat[perm]
    pos = jnp.zeros((TK,), jnp.int32).at[perm].set(iota)
    token_ids = perm // TOPK
    w_sorted = w_flat[perm]
    offsets = jnp.searchsorted(
        e_sorted, jnp.arange(E + 1, dtype=jnp.int32)).astype(jnp.int32)
    counts = offsets[1:] - offsets[:E]

    b0 = offsets[:E] // BLK
    b1 = jnp.maximum(offsets[1:] - 1, 0) // BLK
    nb_e = jnp.where(counts > 0, b1 - b0 + 1, 0)
    cum = jnp.cumsum(nb_e)
    gs = jnp.arange(G, dtype=jnp.int32)
    eids = jnp.searchsorted(cum, gs, side="right").astype(jnp.int32)
    valids = (gs < cum[E - 1]).astype(jnp.int32)
    eids = jnp.clip(eids, 0, E - 1)
    start = jnp.concatenate([jnp.zeros((1,), jnp.int32),
                             cum.astype(jnp.int32)])[eids]
    bids = b0[eids] + (gs - start)
    bids = jnp.where(valids > 0, bids, NB - 1).astype(jnp.int32)

    gather = _make_sc_gather(TK, DIM, 32, 16)
    xs = gather(x, token_ids)

    w2 = w_sorted.reshape(TK, 1)

    grid_spec = pltpu.PrefetchScalarGridSpec(
        num_scalar_prefetch=4,
        grid=(G,),
        in_specs=[
            pl.BlockSpec((BLK, DIM), lambda g, bids, eids, valids, offs: (bids[g], 0)),
            pl.BlockSpec((None, INTER, DIM), lambda g, bids, eids, valids, offs: (eids[g], 0, 0)),
            pl.BlockSpec((None, INTER, DIM), lambda g, bids, eids, valids, offs: (eids[g], 0, 0)),
            pl.BlockSpec((None, DIM, INTER), lambda g, bids, eids, valids, offs: (eids[g], 0, 0)),
            pl.BlockSpec((BLK, 1), lambda g, bids, eids, valids, offs: (bids[g], 0)),
        ],
        out_specs=pl.BlockSpec((BLK, DIM), lambda g, bids, eids, valids, offs: (bids[g], 0)),
    )
    ys = pl.pallas_call(
        functools.partial(_grouped_ffn_body, blk=BLK, inter=INTER),
        grid_spec=grid_spec,
        out_shape=jax.ShapeDtypeStruct((TK, DIM), jnp.float32),
        compiler_params=pltpu.CompilerParams(
            dimension_semantics=("arbitrary",),
        ),
    )(bids, eids, valids, offsets, xs, gate_proj_weight, up_proj_weight,
      down_proj_weight, w2)

    combine = _make_sc_combine(T, DIM, TOPK, 32, 8)
    out = combine(ys, pos)
    return out
